# baseline (speedup 1.0000x reference)
import functools

import jax
import jax.numpy as jnp
from jax.experimental import pallas as pl
from jax.experimental.pallas import tpu as pltpu

_ROWS = 256


def _dma_body(batch, n_chunks, w_hbm, o_hbm, buf, rsem, wsem):
    for k in range(n_chunks):
        pltpu.make_async_copy(
            w_hbm.at[pl.ds(k * _ROWS, _ROWS), :],
            buf.at[pl.ds(k * _ROWS, _ROWS), :],
            rsem,
        ).start()
    for k in range(n_chunks):
        pltpu.make_async_copy(
            w_hbm.at[pl.ds(k * _ROWS, _ROWS), :],
            buf.at[pl.ds(k * _ROWS, _ROWS), :],
            rsem,
        ).wait()
    for b in range(batch):
        pltpu.make_async_copy(buf, o_hbm.at[b], wsem).start()
    for b in range(batch):
        pltpu.make_async_copy(buf, o_hbm.at[b], wsem).wait()


def kernel(tokens, W_pos):
    B, S = tokens.shape
    D = W_pos.shape[1]
    n_chunks = S // _ROWS

    return pl.pallas_call(
        functools.partial(_dma_body, B, n_chunks),
        in_specs=[pl.BlockSpec(memory_space=pl.ANY)],
        out_specs=pl.BlockSpec(memory_space=pl.ANY),
        out_shape=jax.ShapeDtypeStruct((B, S, D), jnp.float32),
        scratch_shapes=(
            [pltpu.VMEM((S, D), jnp.float32)]
            + [pltpu.SemaphoreType.DMA] * 2
        ),
    )(W_pos)
